# baseline (speedup 1.0000x reference)
import functools

import jax
import jax.numpy as jnp
from jax import lax
from jax.experimental import pallas as pl
from jax.experimental.pallas import tpu as pltpu
from jax.experimental.pallas import tpu_sc as plsc

NUM_CLASSES = 1000
BATCH = 16384

NC = 2
NS = 16
L = 16
NW = NC * NS

ROWS_PER_W = BATCH // NW
SLAB = ROWS_PER_W * NUM_CLASSES
ZFILL = SLAB // NS
VECS = ROWS_PER_W // L
SCAT = 128


def _one_hot_body(x_hbm, out_hbm, idx_v, zbuf, widx, ones_v,
                  shared_z, sem_fill, sem_out, sem_sc):
    cid = lax.axis_index("c")
    sid = lax.axis_index("s")
    wid = sid * NC + cid
    rbase = wid * ROWS_PER_W

    idx_cp = pltpu.async_copy(
        x_hbm.at[pl.ds(rbase, ROWS_PER_W)], idx_v, sem_sc)

    zeros16 = jnp.zeros((L,), jnp.float32)
    ones16 = jnp.ones((L,), jnp.float32)
    lane_k = lax.iota(jnp.int32, L) * NUM_CLASSES

    def _zero_chunk(r, carry):
        for k in range(16):
            zbuf[pl.ds(r * 256 + k * L, L)] = zeros16
        return carry

    lax.fori_loop(0, ZFILL // 256, _zero_chunk, 0)
    for k in range(SCAT // L):
        ones_v[pl.ds(k * L, L)] = ones16

    fill_cp = pltpu.async_copy(
        zbuf, shared_z.at[pl.ds(sid * ZFILL, ZFILL)], sem_fill)

    idx_cp.wait()
    for i in range(VECS):
        v = idx_v[pl.ds(i * L, L)]
        flat = (rbase + i * L) * NUM_CLASSES + lane_k + v
        widx[i // 8, pl.ds((i % 8) * L, L)] = flat

    fill_cp.wait()
    plsc.subcore_barrier()

    zero_cp = pltpu.async_copy(
        shared_z, out_hbm.at[pl.ds(rbase * NUM_CLASSES, SLAB)], sem_out)
    zero_cp.wait()

    pend = []
    for j in range(ROWS_PER_W // SCAT):
        pend.append(pltpu.async_copy(
            ones_v, out_hbm.at[widx.at[j]], sem_sc))
    for cp in pend:
        cp.wait()


_one_hot_sc = functools.partial(
    pl.kernel,
    out_type=jax.ShapeDtypeStruct((BATCH * NUM_CLASSES,), jnp.float32),
    mesh=plsc.VectorSubcoreMesh(core_axis_name="c", subcore_axis_name="s"),
    compiler_params=pltpu.CompilerParams(needs_layout_passes=False),
    scratch_types=[
        pltpu.VMEM((ROWS_PER_W,), jnp.int32),
        pltpu.VMEM((ZFILL,), jnp.float32),
        pltpu.VMEM((ROWS_PER_W // SCAT, SCAT), jnp.int32),
        pltpu.VMEM((SCAT,), jnp.float32),
        pltpu.VMEM_SHARED((SLAB,), jnp.float32),
        pltpu.SemaphoreType.DMA,
        pltpu.SemaphoreType.DMA,
        pltpu.SemaphoreType.DMA,
    ],
)(_one_hot_body)


def kernel(x):
    out = _one_hot_sc(jnp.reshape(x, (BATCH,)))
    return jnp.reshape(out, (BATCH, NUM_CLASSES))
